# Initial kernel scaffold; baseline (speedup 1.0000x reference)
import jax
import jax.numpy as jnp
from jax.experimental import pallas as pl


def kernel(matrix, vector, ref):
    raise NotImplementedError("write your pallas kernel here")


# speedup vs baseline: 1.0314x; 1.0314x over previous
import functools

import jax
import jax.numpy as jnp
from jax.experimental import pallas as pl

N = 16384
BR = 128
NR = N // BR


def _body(m_ref, v_ref, r_ref, out_ref, loss_ref, max_ref):
    i = pl.program_id(0)
    m = m_ref[...]
    v = v_ref[...]
    row = jnp.sum(m * v, axis=1)
    out_ref[0, :] = row
    err = row - r_ref[0, :]
    s = jnp.sum(err * err, keepdims=True).reshape(1, 1)
    a = jnp.max(jnp.abs(err), keepdims=True).reshape(1, 1)

    @pl.when(i == 0)
    def _init():
        loss_ref[...] = s
        max_ref[...] = a

    @pl.when(i > 0)
    def _acc():
        loss_ref[...] = loss_ref[...] + s
        max_ref[...] = jnp.maximum(max_ref[...], a)

    @pl.when(i == NR - 1)
    def _fin():
        loss_ref[...] = loss_ref[...] * (1.0 / N)


@functools.partial(jax.jit)
def _run(matrix, vector, ref):
    v2 = vector.reshape(1, N)
    r2 = ref.reshape(1, N)
    out, loss, mabs = pl.pallas_call(
        _body,
        grid=(NR,),
        in_specs=[
            pl.BlockSpec((BR, N), lambda i: (i, 0)),
            pl.BlockSpec((1, N), lambda i: (0, 0)),
            pl.BlockSpec((1, BR), lambda i: (0, i)),
        ],
        out_specs=[
            pl.BlockSpec((1, BR), lambda i: (0, i)),
            pl.BlockSpec((1, 1), lambda i: (0, 0)),
            pl.BlockSpec((1, 1), lambda i: (0, 0)),
        ],
        out_shape=[
            jax.ShapeDtypeStruct((1, N), jnp.float32),
            jax.ShapeDtypeStruct((1, 1), jnp.float32),
            jax.ShapeDtypeStruct((1, 1), jnp.float32),
        ],
    )(matrix, v2, r2)
    return loss[0, 0], out.reshape(N), mabs[0, 0]


def kernel(matrix, vector, ref):
    loss, out, mabs = _run(matrix, vector, ref)
    return (loss, out, ref, mabs)


# speedup vs baseline: 1.0360x; 1.0044x over previous
import functools

import jax
import jax.numpy as jnp
from jax.experimental import pallas as pl

N = 16384
BR = 128
NR = N // BR


def _body(ml_ref, mr_ref, v_ref, r_ref, out_ref, loss_ref, max_ref):
    i = pl.program_id(0)
    v = v_ref[...]
    row = jnp.sum(ml_ref[...] * v[:, : N // 2], axis=1)
    row = row + jnp.sum(mr_ref[...] * v[:, N // 2 :], axis=1)
    out_ref[0, :] = row
    err = row - r_ref[0, :]
    s = jnp.sum(err * err, keepdims=True).reshape(1, 1)
    a = jnp.max(jnp.abs(err), keepdims=True).reshape(1, 1)

    @pl.when(i == 0)
    def _init():
        loss_ref[...] = s
        max_ref[...] = a

    @pl.when(i > 0)
    def _acc():
        loss_ref[...] = loss_ref[...] + s
        max_ref[...] = jnp.maximum(max_ref[...], a)

    @pl.when(i == NR - 1)
    def _fin():
        loss_ref[...] = loss_ref[...] * (1.0 / N)


@functools.partial(jax.jit)
def _run(matrix, vector, ref):
    v2 = vector.reshape(1, N)
    r2 = ref.reshape(1, N)
    out, loss, mabs = pl.pallas_call(
        _body,
        grid=(NR,),
        in_specs=[
            pl.BlockSpec((BR, N // 2), lambda i: (i, 0)),
            pl.BlockSpec((BR, N // 2), lambda i: (i, 1)),
            pl.BlockSpec((1, N), lambda i: (0, 0)),
            pl.BlockSpec((1, BR), lambda i: (0, i)),
        ],
        out_specs=[
            pl.BlockSpec((1, BR), lambda i: (0, i)),
            pl.BlockSpec((1, 1), lambda i: (0, 0)),
            pl.BlockSpec((1, 1), lambda i: (0, 0)),
        ],
        out_shape=[
            jax.ShapeDtypeStruct((1, N), jnp.float32),
            jax.ShapeDtypeStruct((1, 1), jnp.float32),
            jax.ShapeDtypeStruct((1, 1), jnp.float32),
        ],
    )(matrix, matrix, v2, r2)
    return loss[0, 0], out.reshape(N), mabs[0, 0]


def kernel(matrix, vector, ref):
    loss, out, mabs = _run(matrix, vector, ref)
    return (loss, out, ref, mabs)


# speedup vs baseline: 1.0371x; 1.0011x over previous
import functools

import jax
import jax.numpy as jnp
from jax.experimental import pallas as pl

N = 16384
BR = 256
NR = N // BR


def _body(m0_ref, m1_ref, m2_ref, m3_ref, v_ref, r_ref, out_ref, loss_ref, max_ref):
    i = pl.program_id(0)
    v = v_ref[...]
    q = N // 4
    row = jnp.sum(m0_ref[...] * v[:, 0 * q : 1 * q], axis=1)
    row = row + jnp.sum(m1_ref[...] * v[:, 1 * q : 2 * q], axis=1)
    row = row + jnp.sum(m2_ref[...] * v[:, 2 * q : 3 * q], axis=1)
    row = row + jnp.sum(m3_ref[...] * v[:, 3 * q : 4 * q], axis=1)
    out_ref[0, :] = row
    err = row - r_ref[0, :]
    s = jnp.sum(err * err, keepdims=True).reshape(1, 1)
    a = jnp.max(jnp.abs(err), keepdims=True).reshape(1, 1)

    @pl.when(i == 0)
    def _init():
        loss_ref[...] = s
        max_ref[...] = a

    @pl.when(i > 0)
    def _acc():
        loss_ref[...] = loss_ref[...] + s
        max_ref[...] = jnp.maximum(max_ref[...], a)

    @pl.when(i == NR - 1)
    def _fin():
        loss_ref[...] = loss_ref[...] * (1.0 / N)


@functools.partial(jax.jit)
def _run(matrix, vector, ref):
    v2 = vector.reshape(1, N)
    r2 = ref.reshape(1, N)
    out, loss, mabs = pl.pallas_call(
        _body,
        grid=(NR,),
        in_specs=[
            pl.BlockSpec((BR, N // 4), lambda i: (i, 0)),
            pl.BlockSpec((BR, N // 4), lambda i: (i, 1)),
            pl.BlockSpec((BR, N // 4), lambda i: (i, 2)),
            pl.BlockSpec((BR, N // 4), lambda i: (i, 3)),
            pl.BlockSpec((1, N), lambda i: (0, 0)),
            pl.BlockSpec((1, BR), lambda i: (0, i)),
        ],
        out_specs=[
            pl.BlockSpec((1, BR), lambda i: (0, i)),
            pl.BlockSpec((1, 1), lambda i: (0, 0)),
            pl.BlockSpec((1, 1), lambda i: (0, 0)),
        ],
        out_shape=[
            jax.ShapeDtypeStruct((1, N), jnp.float32),
            jax.ShapeDtypeStruct((1, 1), jnp.float32),
            jax.ShapeDtypeStruct((1, 1), jnp.float32),
        ],
    )(matrix, matrix, matrix, matrix, v2, r2)
    return loss[0, 0], out.reshape(N), mabs[0, 0]


def kernel(matrix, vector, ref):
    loss, out, mabs = _run(matrix, vector, ref)
    return (loss, out, ref, mabs)
